# baseline (speedup 1.0000x reference)
import functools

import jax
import jax.numpy as jnp
from jax import lax
from jax.experimental import pallas as pl
from jax.experimental.pallas import tpu as pltpu
from jax.experimental.pallas import tpu_sc as plsc

N = 10000
NPAD = 10240
E = 320000
D = 128
NC = 2
NS = 16
NW = NC * NS
EPW = E // NW
RCHUNK = 125
CHUNK = 128
NCHUNK = EPW // RCHUNK
TRASH = NPAD - 8
BLK = 8
NBLK = NCHUNK // BLK
RPT = NPAD // NS
ZROWS = 32
BM = 1280
NB = NPAD // BM
VL = 16


def _mesh():
    return plsc.VectorSubcoreMesh(core_axis_name="c", subcore_axis_name="s")


@functools.partial(
    pl.kernel,
    out_type=jax.ShapeDtypeStruct((NC * NPAD, D), jnp.float32),
    mesh=_mesh(),
    scratch_types=[
        pltpu.VMEM((NCHUNK, CHUNK), jnp.int32),
        pltpu.VMEM((CHUNK, D), jnp.float32),
        pltpu.VMEM((ZROWS, D), jnp.float32),
        pltpu.VMEM_SHARED((NPAD, D), jnp.float32),
    ],
)
def _deg_kernel(col_hbm, degp_hbm, cidx_v, ones_v, zbuf_v, acc_sh):
    c = lax.axis_index("c")
    s = lax.axis_index("s")
    wid = s * NC + c

    pltpu.sync_copy(col_hbm.at[wid], cidx_v)

    zeros16 = jnp.zeros((VL,), jnp.float32)
    ones16 = jnp.ones((VL,), jnp.float32)

    def fill_z(i, carry):
        for j in range(D // VL):
            zbuf_v[i, pl.ds(j * VL, VL)] = zeros16
        return carry

    lax.fori_loop(0, ZROWS, fill_z, 0)

    def fill_o(i, carry):
        for j in range(D // VL):
            ones_v[i, pl.ds(j * VL, VL)] = ones16
        return carry

    lax.fori_loop(0, CHUNK, fill_o, 0)

    r0 = s * RPT
    for i in range(RPT // ZROWS):
        pltpu.sync_copy(zbuf_v, acc_sh.at[pl.ds(r0 + i * ZROWS, ZROWS)])
    plsc.subcore_barrier()

    def chunk(j, carry):
        pltpu.sync_copy(ones_v, acc_sh.at[cidx_v.at[j]], add=True)
        return carry

    lax.fori_loop(0, NCHUNK, chunk, 0)
    plsc.subcore_barrier()

    pltpu.sync_copy(acc_sh.at[pl.ds(r0, RPT)],
                    degp_hbm.at[pl.ds(c * NPAD + r0, RPT)])


@functools.partial(
    pl.kernel,
    out_type=jax.ShapeDtypeStruct((NC * NPAD, D), jnp.float32),
    mesh=_mesh(),
    scratch_types=[
        pltpu.VMEM((2, BLK, CHUNK), jnp.int32),
        pltpu.VMEM((2, BLK, CHUNK), jnp.int32),
        pltpu.VMEM((CHUNK, D), jnp.float32),
        pltpu.VMEM((CHUNK, D), jnp.float32),
        pltpu.VMEM((ZROWS, D), jnp.float32),
        pltpu.VMEM_SHARED((NPAD, D), jnp.float32),
        pltpu.SemaphoreType.DMA,
        pltpu.SemaphoreType.DMA,
        pltpu.SemaphoreType.DMA,
        pltpu.SemaphoreType.DMA,
    ],
)
def _agg_kernel(y_hbm, row_hbm, col_hbm, aggp_hbm,
                rblk_v, cblk_v, buf0_v, buf1_v, zbuf_v, acc_sh,
                gsem0, gsem1, isem0, isem1):
    c = lax.axis_index("c")
    s = lax.axis_index("s")
    wid = s * NC + c

    bufs = (buf0_v, buf1_v)
    gsems = (gsem0, gsem1)
    isems = (isem0, isem1)

    def _fetch(g, slot):
        pltpu.async_copy(row_hbm.at[wid, g], rblk_v.at[slot], isems[slot])
        pltpu.async_copy(col_hbm.at[wid, g], cblk_v.at[slot], isems[slot])

    def _wait_fetch(g, slot):
        pltpu.make_async_copy(row_hbm.at[wid, g], rblk_v.at[slot],
                              isems[slot]).wait()
        pltpu.make_async_copy(col_hbm.at[wid, g], cblk_v.at[slot],
                              isems[slot]).wait()

    def _gather(slot, r, b):
        pltpu.async_copy(y_hbm.at[rblk_v.at[slot, r]], bufs[b], gsems[b])

    def _wait_gather(slot, r, b):
        pltpu.make_async_copy(y_hbm.at[rblk_v.at[slot, r]], bufs[b],
                              gsems[b]).wait()

    def _scatter(slot, r, b):
        pltpu.sync_copy(bufs[b], acc_sh.at[cblk_v.at[slot, r]], add=True)

    pltpu.sync_copy(row_hbm.at[wid, 0], rblk_v.at[0])
    pltpu.sync_copy(col_hbm.at[wid, 0], cblk_v.at[0])
    _gather(0, 0, 0)
    _gather(0, 1, 1)

    zeros16 = jnp.zeros((VL,), jnp.float32)

    def fill_z(i, carry):
        for j in range(D // VL):
            zbuf_v[i, pl.ds(j * VL, VL)] = zeros16
        return carry

    lax.fori_loop(0, ZROWS, fill_z, 0)

    r0 = s * RPT
    for i in range(RPT // ZROWS):
        pltpu.sync_copy(zbuf_v, acc_sh.at[pl.ds(r0 + i * ZROWS, ZROWS)])
    plsc.subcore_barrier()

    def blockstep(g, slot, last_block=False):
        nslot = 1 - slot
        if not last_block:
            _fetch(g + 1, nslot)
        for b in range(BLK):
            if b == 6 and not last_block:
                _wait_fetch(g + 1, nslot)
            _wait_gather(slot, b, b % 2)
            _scatter(slot, b, b % 2)
            if b < 6:
                _gather(slot, b + 2, b % 2)
            elif not last_block:
                _gather(nslot, b - 6, b % 2)

    blockstep(0, 0)

    def pairstep(h, carry):
        g1 = 2 * h + 1
        blockstep(g1, 1)
        blockstep(g1 + 1, 0)
        return carry

    lax.fori_loop(0, (NBLK - 2) // 2, pairstep, 0)
    blockstep(NBLK - 1, 1, last_block=True)
    plsc.subcore_barrier()

    pltpu.sync_copy(acc_sh.at[pl.ds(r0, RPT)],
                    aggp_hbm.at[pl.ds(c * NPAD + r0, RPT)])


def _dis(d0_ref, d1_ref):
    return lax.rsqrt(1.0 + d0_ref[:, 0:1] + d1_ref[:, 0:1])


def _mm_scale_body(x_ref, w_ref, d0_ref, d1_ref, y_ref):
    dis = _dis(d0_ref, d1_ref)
    y_ref[...] = jnp.dot(x_ref[...], w_ref[...],
                         preferred_element_type=jnp.float32) * dis


def _tc_layer1(x, W1, degp):
    return pl.pallas_call(
        _mm_scale_body,
        grid=(NB,),
        in_specs=[
            pl.BlockSpec((BM, D), lambda i: (i, 0)),
            pl.BlockSpec((D, D), lambda i: (0, 0)),
            pl.BlockSpec((BM, D), lambda i: (i, 0)),
            pl.BlockSpec((BM, D), lambda i: (NB + i, 0)),
        ],
        out_specs=pl.BlockSpec((BM, D), lambda i: (i, 0)),
        out_shape=jax.ShapeDtypeStruct((NPAD, D), jnp.float32),
    )(x, W1, degp, degp)


def _combine_mm_body(a0_ref, a1_ref, y1_ref, d0_ref, d1_ref, w_ref, b_ref,
                     y2_ref):
    dis = _dis(d0_ref, d1_ref)
    h = jnp.maximum(
        dis * (a0_ref[...] + a1_ref[...] + y1_ref[...]) + b_ref[...], 0.0)
    y2_ref[...] = jnp.dot(h, w_ref[...],
                          preferred_element_type=jnp.float32) * dis


def _tc_layer2(aggp, y1, degp, W2, b1):
    return pl.pallas_call(
        _combine_mm_body,
        grid=(NB,),
        in_specs=[
            pl.BlockSpec((BM, D), lambda i: (i, 0)),
            pl.BlockSpec((BM, D), lambda i: (NB + i, 0)),
            pl.BlockSpec((BM, D), lambda i: (i, 0)),
            pl.BlockSpec((BM, D), lambda i: (i, 0)),
            pl.BlockSpec((BM, D), lambda i: (NB + i, 0)),
            pl.BlockSpec((D, D), lambda i: (0, 0)),
            pl.BlockSpec((1, D), lambda i: (0, 0)),
        ],
        out_specs=pl.BlockSpec((BM, D), lambda i: (i, 0)),
        out_shape=jax.ShapeDtypeStruct((NPAD, D), jnp.float32),
    )(aggp, aggp, y1, degp, degp, W2, b1)


def _final_body(a0_ref, a1_ref, y2_ref, d0_ref, d1_ref, b_ref, o_ref):
    dis = _dis(d0_ref, d1_ref)
    o_ref[...] = jnp.maximum(
        dis * (a0_ref[...] + a1_ref[...] + y2_ref[...]) + b_ref[...], 0.0)


def _tc_final(aggp, y2, degp, b2):
    return pl.pallas_call(
        _final_body,
        grid=(NB,),
        in_specs=[
            pl.BlockSpec((BM, D), lambda i: (i, 0)),
            pl.BlockSpec((BM, D), lambda i: (NB + i, 0)),
            pl.BlockSpec((BM, D), lambda i: (i, 0)),
            pl.BlockSpec((BM, D), lambda i: (i, 0)),
            pl.BlockSpec((BM, D), lambda i: (NB + i, 0)),
            pl.BlockSpec((1, D), lambda i: (0, 0)),
        ],
        out_specs=pl.BlockSpec((BM, D), lambda i: (i, 0)),
        out_shape=jax.ShapeDtypeStruct((NPAD, D), jnp.float32),
    )(aggp, aggp, y2, degp, degp, b2)


def kernel(x, edge_index, W1, b1, W2, b2):
    ei = edge_index.astype(jnp.int32)
    row = ei[0].reshape(NW, NCHUNK, RCHUNK)
    col = ei[1].reshape(NW, NCHUNK, RCHUNK)
    row = jnp.pad(row, ((0, 0), (0, 0), (0, CHUNK - RCHUNK)))
    col = jnp.pad(col, ((0, 0), (0, 0), (0, CHUNK - RCHUNK)),
                  constant_values=TRASH)
    row4 = row.reshape(NW, NBLK, BLK, CHUNK)
    col4 = col.reshape(NW, NBLK, BLK, CHUNK)
    xp = jnp.pad(x, ((0, NPAD - N), (0, 0)))
    degp = _deg_kernel(col)
    y1 = _tc_layer1(xp, W1, degp)
    a1 = _agg_kernel(y1, row4, col4)
    y2 = _tc_layer2(a1, y1, degp, W2, b1.reshape(1, D))
    a2 = _agg_kernel(y2, row4, col4)
    return _tc_final(a2, y2, degp, b2.reshape(1, D))[:N]


# speedup vs baseline: 2.4425x; 2.4425x over previous
import functools

import jax
import jax.numpy as jnp
from jax import lax
from jax.experimental import pallas as pl
from jax.experimental.pallas import tpu as pltpu
from jax.experimental.pallas import tpu_sc as plsc

N = 10000
NPAD = 10240
E = 320000
D = 128
NC = 2
NS = 16
NW = NC * NS
EPW = E // NW
CHUNK = 80
NCHUNK = EPW // CHUNK
BLK = 5
NBLK = NCHUNK // BLK
RPT = NPAD // NS
ZROWS = 32
BM = 1280
NB = NPAD // BM
VL = 16


def _mesh():
    return plsc.VectorSubcoreMesh(core_axis_name="c", subcore_axis_name="s")


@functools.partial(
    pl.kernel,
    out_type=jax.ShapeDtypeStruct((NC * NPAD, D), jnp.float32),
    mesh=_mesh(),
    scratch_types=[
        pltpu.VMEM((NCHUNK, CHUNK), jnp.int32),
        pltpu.VMEM((CHUNK, D), jnp.float32),
        pltpu.VMEM((ZROWS, D), jnp.float32),
        pltpu.VMEM_SHARED((NPAD, D), jnp.float32),
    ],
)
def _deg_kernel(col_hbm, degp_hbm, cidx_v, ones_v, zbuf_v, acc_sh):
    c = lax.axis_index("c")
    s = lax.axis_index("s")
    wid = s * NC + c

    pltpu.sync_copy(col_hbm.at[wid], cidx_v)

    zeros16 = jnp.zeros((VL,), jnp.float32)
    ones16 = jnp.ones((VL,), jnp.float32)

    def fill_z(i, carry):
        for j in range(D // VL):
            zbuf_v[i, pl.ds(j * VL, VL)] = zeros16
        return carry

    lax.fori_loop(0, ZROWS, fill_z, 0)

    def fill_o(i, carry):
        for j in range(D // VL):
            ones_v[i, pl.ds(j * VL, VL)] = ones16
        return carry

    lax.fori_loop(0, CHUNK, fill_o, 0)

    r0 = s * RPT
    for i in range(RPT // ZROWS):
        pltpu.sync_copy(zbuf_v, acc_sh.at[pl.ds(r0 + i * ZROWS, ZROWS)])
    plsc.subcore_barrier()

    def chunk(j, carry):
        pltpu.sync_copy(ones_v, acc_sh.at[cidx_v.at[j]], add=True)
        return carry

    lax.fori_loop(0, NCHUNK, chunk, 0)
    plsc.subcore_barrier()

    pltpu.sync_copy(acc_sh.at[pl.ds(r0, RPT)],
                    degp_hbm.at[pl.ds(c * NPAD + r0, RPT)])


@functools.partial(
    pl.kernel,
    out_type=jax.ShapeDtypeStruct((NC * NPAD, D), jnp.float32),
    mesh=_mesh(),
    scratch_types=[
        pltpu.VMEM((2, BLK, CHUNK), jnp.int32),
        pltpu.VMEM((2, BLK, CHUNK), jnp.int32),
        pltpu.VMEM((CHUNK, D), jnp.float32),
        pltpu.VMEM((CHUNK, D), jnp.float32),
        pltpu.VMEM((ZROWS, D), jnp.float32),
        pltpu.VMEM_SHARED((NPAD, D), jnp.float32),
        pltpu.SemaphoreType.DMA,
        pltpu.SemaphoreType.DMA,
        pltpu.SemaphoreType.DMA,
        pltpu.SemaphoreType.DMA,
    ],
)
def _agg_kernel(y_hbm, row_hbm, col_hbm, aggp_hbm,
                rblk_v, cblk_v, buf0_v, buf1_v, zbuf_v, acc_sh,
                gsem0, gsem1, isem0, isem1):
    c = lax.axis_index("c")
    s = lax.axis_index("s")
    wid = s * NC + c

    bufs = (buf0_v, buf1_v)
    gsems = (gsem0, gsem1)
    isems = (isem0, isem1)

    def _fetch(g, slot):
        pltpu.async_copy(row_hbm.at[wid, g], rblk_v.at[slot], isems[slot])
        pltpu.async_copy(col_hbm.at[wid, g], cblk_v.at[slot], isems[slot])

    def _wait_fetch(g, slot):
        pltpu.make_async_copy(row_hbm.at[wid, g], rblk_v.at[slot],
                              isems[slot]).wait()
        pltpu.make_async_copy(col_hbm.at[wid, g], cblk_v.at[slot],
                              isems[slot]).wait()

    def _gather(slot, r, b):
        pltpu.async_copy(y_hbm.at[rblk_v.at[slot, r]], bufs[b], gsems[b])

    def _wait_gather(slot, r, b):
        pltpu.make_async_copy(y_hbm.at[rblk_v.at[slot, r]], bufs[b],
                              gsems[b]).wait()

    def _scatter(slot, r, b):
        pltpu.sync_copy(bufs[b], acc_sh.at[cblk_v.at[slot, r]], add=True)

    pltpu.sync_copy(row_hbm.at[wid, 0], rblk_v.at[0])
    pltpu.sync_copy(col_hbm.at[wid, 0], cblk_v.at[0])
    _gather(0, 0, 0)
    _gather(0, 1, 1)

    zeros16 = jnp.zeros((VL,), jnp.float32)

    def fill_z(i, carry):
        for j in range(D // VL):
            zbuf_v[i, pl.ds(j * VL, VL)] = zeros16
        return carry

    lax.fori_loop(0, ZROWS, fill_z, 0)

    r0 = s * RPT
    for i in range(RPT // ZROWS):
        pltpu.sync_copy(zbuf_v, acc_sh.at[pl.ds(r0 + i * ZROWS, ZROWS)])
    plsc.subcore_barrier()

    def blockstep(g, slot, gp, last_block=False):
        nslot = 1 - slot
        if not last_block:
            _fetch(g + 1, nslot)
        for b in range(BLK):
            if b == 3 and not last_block:
                _wait_fetch(g + 1, nslot)
            _wait_gather(slot, b, (gp + b) % 2)
            _scatter(slot, b, (gp + b) % 2)
            if b < 3:
                _gather(slot, b + 2, (gp + b) % 2)
            elif not last_block:
                _gather(nslot, b - 3, (gp + b) % 2)

    blockstep(0, 0, 0)

    def pairstep(h, carry):
        g1 = 2 * h + 1
        blockstep(g1, 1, 1)
        blockstep(g1 + 1, 0, 0)
        return carry

    lax.fori_loop(0, (NBLK - 3) // 2, pairstep, 0)
    blockstep(NBLK - 2, 1, 1)
    blockstep(NBLK - 1, 0, 0, last_block=True)
    plsc.subcore_barrier()

    pltpu.sync_copy(acc_sh.at[pl.ds(r0, RPT)],
                    aggp_hbm.at[pl.ds(c * NPAD + r0, RPT)])


def _dis(d0_ref, d1_ref):
    return lax.rsqrt(1.0 + d0_ref[:, 0:1] + d1_ref[:, 0:1])


def _mm_scale_body(x_ref, w_ref, d0_ref, d1_ref, y_ref):
    dis = _dis(d0_ref, d1_ref)
    y_ref[...] = jnp.dot(x_ref[...], w_ref[...],
                         preferred_element_type=jnp.float32) * dis


def _tc_layer1(x, W1, degp):
    return pl.pallas_call(
        _mm_scale_body,
        grid=(NB,),
        in_specs=[
            pl.BlockSpec((BM, D), lambda i: (i, 0)),
            pl.BlockSpec((D, D), lambda i: (0, 0)),
            pl.BlockSpec((BM, D), lambda i: (i, 0)),
            pl.BlockSpec((BM, D), lambda i: (NB + i, 0)),
        ],
        out_specs=pl.BlockSpec((BM, D), lambda i: (i, 0)),
        out_shape=jax.ShapeDtypeStruct((NPAD, D), jnp.float32),
    )(x, W1, degp, degp)


def _combine_mm_body(a0_ref, a1_ref, y1_ref, d0_ref, d1_ref, w_ref, b_ref,
                     y2_ref):
    dis = _dis(d0_ref, d1_ref)
    h = jnp.maximum(
        dis * (a0_ref[...] + a1_ref[...] + y1_ref[...]) + b_ref[...], 0.0)
    y2_ref[...] = jnp.dot(h, w_ref[...],
                          preferred_element_type=jnp.float32) * dis


def _tc_layer2(aggp, y1, degp, W2, b1):
    return pl.pallas_call(
        _combine_mm_body,
        grid=(NB,),
        in_specs=[
            pl.BlockSpec((BM, D), lambda i: (i, 0)),
            pl.BlockSpec((BM, D), lambda i: (NB + i, 0)),
            pl.BlockSpec((BM, D), lambda i: (i, 0)),
            pl.BlockSpec((BM, D), lambda i: (i, 0)),
            pl.BlockSpec((BM, D), lambda i: (NB + i, 0)),
            pl.BlockSpec((D, D), lambda i: (0, 0)),
            pl.BlockSpec((1, D), lambda i: (0, 0)),
        ],
        out_specs=pl.BlockSpec((BM, D), lambda i: (i, 0)),
        out_shape=jax.ShapeDtypeStruct((NPAD, D), jnp.float32),
    )(aggp, aggp, y1, degp, degp, W2, b1)


def _final_body(a0_ref, a1_ref, y2_ref, d0_ref, d1_ref, b_ref, o_ref):
    dis = _dis(d0_ref, d1_ref)
    o_ref[...] = jnp.maximum(
        dis * (a0_ref[...] + a1_ref[...] + y2_ref[...]) + b_ref[...], 0.0)


def _tc_final(aggp, y2, degp, b2):
    return pl.pallas_call(
        _final_body,
        grid=(NB,),
        in_specs=[
            pl.BlockSpec((BM, D), lambda i: (i, 0)),
            pl.BlockSpec((BM, D), lambda i: (NB + i, 0)),
            pl.BlockSpec((BM, D), lambda i: (i, 0)),
            pl.BlockSpec((BM, D), lambda i: (i, 0)),
            pl.BlockSpec((BM, D), lambda i: (NB + i, 0)),
            pl.BlockSpec((1, D), lambda i: (0, 0)),
        ],
        out_specs=pl.BlockSpec((BM, D), lambda i: (i, 0)),
        out_shape=jax.ShapeDtypeStruct((NPAD, D), jnp.float32),
    )(aggp, aggp, y2, degp, degp, b2)


def kernel(x, edge_index, W1, b1, W2, b2):
    ei = edge_index.astype(jnp.int32)
    row = ei[0].reshape(NW, NCHUNK, CHUNK)
    col = ei[1].reshape(NW, NCHUNK, CHUNK)
    row4 = row.reshape(NW, NBLK, BLK, CHUNK)
    col4 = col.reshape(NW, NBLK, BLK, CHUNK)
    xp = jnp.pad(x, ((0, NPAD - N), (0, 0)))
    degp = _deg_kernel(col)
    y1 = _tc_layer1(xp, W1, degp)
    a1 = _agg_kernel(y1, row4, col4)
    y2 = _tc_layer2(a1, y1, degp, W2, b1.reshape(1, D))
    a2 = _agg_kernel(y2, row4, col4)
    return _tc_final(a2, y2, degp, b2.reshape(1, D))[:N]
